# baseline (speedup 1.0000x reference)
import functools

import jax
import jax.numpy as jnp
from jax import lax
from jax.experimental import pallas as pl
from jax.experimental.pallas import tpu as pltpu
from jax.experimental.pallas import tpu_sc as plsc

_NUM_CORES = 2
_NUM_SUBCORES = 16
_NUM_WORKERS = _NUM_CORES * _NUM_SUBCORES


@functools.lru_cache(maxsize=None)
def _build(seq_len: int, d_model: int):
    assert seq_len % _NUM_WORKERS == 0
    rows_per_worker = seq_len // _NUM_WORKERS
    chunk = min(16, rows_per_worker)
    n_chunks = rows_per_worker // chunk
    n_buf = min(6, n_chunks)

    mesh = plsc.VectorSubcoreMesh(
        core_axis_name="c", subcore_axis_name="s", num_cores=_NUM_CORES
    )

    @functools.partial(
        pl.kernel,
        mesh=mesh,
        out_type=jax.ShapeDtypeStruct((seq_len, d_model), jnp.float32),
        scratch_types=[
            [pltpu.VMEM((chunk, d_model), jnp.float32) for _ in range(n_buf)],
            [pltpu.SemaphoreType.DMA for _ in range(n_buf)],
            [pltpu.SemaphoreType.DMA for _ in range(n_buf)],
        ],
    )
    def copy_rows(table_hbm, out_hbm, bufs, gsems, ssems):
        wid = lax.axis_index("s") * _NUM_CORES + lax.axis_index("c")

        def src(i):
            return table_hbm.at[pl.ds((i * _NUM_WORKERS + wid) * chunk, chunk)]

        def dst(i):
            return out_hbm.at[pl.ds((i * _NUM_WORKERS + wid) * chunk, chunk)]

        gp = [None] * n_buf
        sp = [None] * n_buf
        for i in range(n_buf):
            gp[i] = pltpu.async_copy(src(i), bufs[i], gsems[i])
        for i in range(n_chunks):
            k = i % n_buf
            gp[k].wait()
            sp[k] = pltpu.async_copy(bufs[k], dst(i), ssems[k])
            j = i + n_buf
            if j < n_chunks:
                sp[k].wait()
                gp[k] = pltpu.async_copy(src(j), bufs[k], gsems[k])
                sp[k] = None
        for p in sp:
            if p is not None:
                p.wait()

    return copy_rows


def kernel(x, table):
    seq_len = x.shape[1]
    out = _build(seq_len, table.shape[1])(table)
    return out[None, :, :]


# speedup vs baseline: 1.0080x; 1.0080x over previous
import functools

import jax
import jax.numpy as jnp
from jax import lax
from jax.experimental import pallas as pl
from jax.experimental.pallas import tpu as pltpu
from jax.experimental.pallas import tpu_sc as plsc

_NUM_CORES = 2
_NUM_SUBCORES = 16
_NUM_WORKERS = _NUM_CORES * _NUM_SUBCORES


@functools.lru_cache(maxsize=None)
def _build(seq_len: int, d_model: int):
    assert seq_len % _NUM_WORKERS == 0
    rows_per_worker = seq_len // _NUM_WORKERS
    chunk = min(16, rows_per_worker)
    n_chunks = rows_per_worker // chunk
    n_buf = min(6, n_chunks)

    mesh = plsc.VectorSubcoreMesh(
        core_axis_name="c", subcore_axis_name="s", num_cores=_NUM_CORES
    )

    @functools.partial(
        pl.kernel,
        mesh=mesh,
        out_type=jax.ShapeDtypeStruct((seq_len, d_model), jnp.float32),
        scratch_types=[
            [pltpu.VMEM((chunk, d_model), jnp.float32) for _ in range(n_buf)],
            [pltpu.SemaphoreType.DMA for _ in range(n_buf)],
            [pltpu.SemaphoreType.DMA for _ in range(n_buf)],
        ],
    )
    def copy_rows(table_hbm, out_hbm, bufs, gsems, ssems):
        wid = lax.axis_index("s") * _NUM_CORES + lax.axis_index("c")
        base = wid * rows_per_worker

        def src(i):
            return table_hbm.at[pl.ds(base + i * chunk, chunk)]

        def dst(i):
            return out_hbm.at[pl.ds(base + i * chunk, chunk)]

        gp = [None] * n_buf
        sp = [None] * n_buf
        for i in range(n_buf):
            gp[i] = pltpu.async_copy(src(i), bufs[i], gsems[i])
        for i in range(n_chunks):
            k = i % n_buf
            gp[k].wait()
            sp[k] = pltpu.async_copy(bufs[k], dst(i), ssems[k])
            j = i + n_buf
            if j < n_chunks:
                sp[k].wait()
                gp[k] = pltpu.async_copy(src(j), bufs[k], gsems[k])
                sp[k] = None
        for p in sp:
            if p is not None:
                p.wait()

    return copy_rows


def kernel(x, table):
    seq_len = x.shape[1]
    out = _build(seq_len, table.shape[1])(table)
    return out[None, :, :]
